# Initial kernel scaffold; baseline (speedup 1.0000x reference)
import jax
import jax.numpy as jnp
from jax.experimental import pallas as pl


def kernel(h, x, v, pairlist, W_in, b_in, W_e1, b_e1, W_e2, b_e2, W_att, b_att, W_mix, W_v, W_n1, b_n1, W_n2, b_n2, W_p1, b_p1, W_p2, b_p2, W_vel1, b_vel1, W_vel2):
    raise NotImplementedError("write your pallas kernel here")


# speedup vs baseline: 6.2867x; 6.2867x over previous
import functools

import jax
import jax.numpy as jnp
from jax import lax
from jax.experimental import pallas as pl
from jax.experimental.pallas import tpu as pltpu
from jax.experimental.pallas import tpu_sc as plsc

N = 10000
E = 160000
D = 128
NRBF = 50
NH = 2
COEFF = 256
CUTOFF = 5.0
EPS = 1e-8
PW = 192
XC = D + NRBF

NC, NS = 2, 16
NW = NC * NS
CH = 200

_SC_MESH = dict(
    mesh=plsc.VectorSubcoreMesh(core_axis_name="c", subcore_axis_name="s"),
    compiler_params=pltpu.CompilerParams(use_tc_tiling_on_sc=False,
                                         needs_layout_passes=False),
)


def _splat(v):
    return jnp.broadcast_to(v, (16,)).astype(jnp.int32)


def _silu(z):
    return z * (1.0 / (1.0 + jnp.exp(-z)))


def _nodeproj_body(h_ref, xpi_ref, xpj_ref, wi_ref, wj_ref, pi_ref, pj_ref):
    h = h_ref[...]
    pi_ref[...] = jnp.dot(h, wi_ref[...], preferred_element_type=jnp.float32) + xpi_ref[...]
    pj_ref[...] = jnp.dot(h, wj_ref[...], preferred_element_type=jnp.float32) + xpj_ref[...]


def _nodeproj(h, xpi, xpj, wi, wj):
    nb = 2000
    return pl.pallas_call(
        _nodeproj_body,
        grid=(N // nb,),
        in_specs=[
            pl.BlockSpec((nb, D), lambda i: (i, 0)),
            pl.BlockSpec((nb, PW), lambda i: (i, 0)),
            pl.BlockSpec((nb, PW), lambda i: (i, 0)),
            pl.BlockSpec((D, PW), lambda i: (0, 0)),
            pl.BlockSpec((D, PW), lambda i: (0, 0)),
        ],
        out_specs=[
            pl.BlockSpec((nb, PW), lambda i: (i, 0)),
            pl.BlockSpec((nb, PW), lambda i: (i, 0)),
        ],
        out_shape=[
            jax.ShapeDtypeStruct((N, PW), jnp.float32),
            jax.ShapeDtypeStruct((N, PW), jnp.float32),
        ],
    )(h, xpi, xpj, wi, wj)


@functools.partial(
    pl.kernel,
    out_type=jax.ShapeDtypeStruct((E, PW), jnp.float32),
    scratch_types=[
        pltpu.VMEM((CH,), jnp.int32),
        pltpu.VMEM((CH,), jnp.int32),
        pltpu.VMEM((CH, PW), jnp.float32),
        pltpu.VMEM((CH, PW), jnp.float32),
        pltpu.SemaphoreType.DMA,
        pltpu.SemaphoreType.DMA,
    ],
    **_SC_MESH,
)
def _edge_gather(pi_hbm, pj_hbm, ii_hbm, jj_hbm, out_hbm, iv, jv, bi, bj, s1, s2):
    wid = lax.axis_index("s") * NC + lax.axis_index("c")
    base = wid * (E // NW)

    def chunk(t, _):
        off = base + t * CH
        pltpu.sync_copy(ii_hbm.at[pl.ds(off, CH)], iv)
        pltpu.sync_copy(jj_hbm.at[pl.ds(off, CH)], jv)
        cp1 = pltpu.async_copy(pi_hbm.at[iv], bi, s1)
        cp2 = pltpu.async_copy(pj_hbm.at[jv], bj, s2)
        cp1.wait()
        cp2.wait()

        def addrow(r, _):
            for cg in range(PW // 16):
                sl = pl.ds(cg * 16, 16)
                bi[r, sl] = bi[r, sl] + bj[r, sl]
            return 0

        lax.fori_loop(0, CH, addrow, 0, unroll=2)
        pltpu.sync_copy(bi, out_hbm.at[pl.ds(off, CH)])
        return 0

    lax.fori_loop(0, (E // NW) // CH, chunk, 0)


def _edge_a_body(elin_ref, we1f_ref, wd_ref, be1_ref, we2_ref, be2_ref,
                 watt_ref, batt_ref, he_ref, ex_ref):
    elin = elin_ref[...]
    eb = elin.shape[0]
    bpart = elin[:, :D]
    apart = elin[:, D:D + NRBF]
    r3 = elin[:, XC:XC + 3]
    d2 = jnp.sum(r3 * r3, axis=1, keepdims=True) + 1e-24
    dd = jnp.sqrt(d2)
    dirv = r3 / (dd + EPS)
    step = CUTOFF / (NRBF - 1)
    centers = step * lax.broadcasted_iota(jnp.int32, (1, NRBF), 1).astype(jnp.float32)
    rbf = jnp.exp(-((dd - centers) ** 2) * (1.0 / (2.0 * step * step)))
    filt = rbf * apart
    e1 = (bpart + jnp.dot(filt, we1f_ref[...], preferred_element_type=jnp.float32)
          + dd * wd_ref[...] + be1_ref[...])
    he = jnp.dot(_silu(e1), we2_ref[...], preferred_element_type=jnp.float32) + be2_ref[...]
    he_ref[...] = he
    aw = jnp.dot(he, watt_ref[...], preferred_element_type=jnp.float32) + batt_ref[...]
    cw = jnp.where(aw > 0, aw, 2.0 * (jnp.exp(aw * 0.5) - 1.0))
    cc = jnp.where(dd < CUTOFF, 0.5 * (jnp.cos(jnp.pi / CUTOFF * dd) + 1.0), 0.0)
    ew = jnp.exp(cw * cc)[:, :2]
    ex_ref[...] = jnp.concatenate(
        [ew, dirv, jnp.ones((eb, 1), jnp.float32), jnp.zeros((eb, 2), jnp.float32)],
        axis=1)


def _edge_a(elin, we1f, wd, be1, we2, be2, watt, batt):
    eb = 2000
    return pl.pallas_call(
        _edge_a_body,
        grid=(E // eb,),
        in_specs=[
            pl.BlockSpec((eb, PW), lambda i: (i, 0)),
            pl.BlockSpec((NRBF, D), lambda i: (0, 0)),
            pl.BlockSpec((1, D), lambda i: (0, 0)),
            pl.BlockSpec((1, D), lambda i: (0, 0)),
            pl.BlockSpec((D, D), lambda i: (0, 0)),
            pl.BlockSpec((1, D), lambda i: (0, 0)),
            pl.BlockSpec((D, 8), lambda i: (0, 0)),
            pl.BlockSpec((1, 8), lambda i: (0, 0)),
        ],
        out_specs=[
            pl.BlockSpec((eb, D), lambda i: (i, 0)),
            pl.BlockSpec((eb, 8), lambda i: (i, 0)),
        ],
        out_shape=[
            jax.ShapeDtypeStruct((E, D), jnp.float32),
            jax.ShapeDtypeStruct((E, 8), jnp.float32),
        ],
    )(elin, we1f, wd, be1, we2, be2, watt, batt)


@functools.partial(
    pl.kernel,
    out_type=[
        jax.ShapeDtypeStruct((N, 8), jnp.float32),
        jax.ShapeDtypeStruct((N, 8), jnp.float32),
    ],
    scratch_types=[
        pltpu.VMEM((CH,), jnp.int32),
        pltpu.VMEM((CH, 8), jnp.float32),
        pltpu.VMEM_SHARED((N, 8), jnp.float32),
    ],
    **_SC_MESH,
)
def _scatter_rows8(rows_hbm, idx_hbm, zeros_hbm, out_a, out_b, iv, buf, acc):
    c = lax.axis_index("c")
    s = lax.axis_index("s")

    @pl.when(s < 10)
    def _():
        pltpu.sync_copy(zeros_hbm.at[pl.ds(s * 1000, 1000), pl.ds(0, 8)],
                        acc.at[pl.ds(s * 1000, 1000)])

    plsc.subcore_barrier()

    base = c * (E // NC) + s * (E // NC // NS)

    def chunk(t, _):
        off = base + t * CH
        pltpu.sync_copy(idx_hbm.at[pl.ds(off, CH)], iv)
        pltpu.sync_copy(rows_hbm.at[pl.ds(off, CH)], buf)
        pltpu.sync_copy(buf, acc.at[iv], add=True)
        return 0

    lax.fori_loop(0, (E // NC // NS) // CH, chunk, 0)
    plsc.subcore_barrier()

    @pl.when(jnp.logical_and(c == 0, s < 10))
    def _():
        pltpu.sync_copy(acc.at[pl.ds(s * 1000, 1000)],
                        out_a.at[pl.ds(s * 1000, 1000)])

    @pl.when(jnp.logical_and(c == 1, s < 10))
    def _():
        pltpu.sync_copy(acc.at[pl.ds(s * 1000, 1000)],
                        out_b.at[pl.ds(s * 1000, 1000)])


@functools.partial(
    pl.kernel,
    out_type=jax.ShapeDtypeStruct((E, 8), jnp.float32),
    scratch_types=[
        pltpu.VMEM((CH,), jnp.int32),
        pltpu.VMEM((CH, 8), jnp.float32),
        pltpu.VMEM((CH, 8), jnp.float32),
        pltpu.VMEM((CH, 8), jnp.float32),
        pltpu.VMEM((CH, 8), jnp.float32),
        pltpu.SemaphoreType.DMA,
        pltpu.SemaphoreType.DMA,
    ],
    **_SC_MESH,
)
def _edge_att(ex_hbm, sa_hbm, sb_hbm, idx_hbm, out_hbm, iv, ebuf, ra, rb, obuf,
              sem1, sem2):
    wid = lax.axis_index("s") * NC + lax.axis_index("c")
    base = wid * (E // NW)
    lane = lax.broadcasted_iota(jnp.int32, (16,), 0)
    ci = lane & 7

    def chunk(t, _):
        off = base + t * CH
        pltpu.sync_copy(idx_hbm.at[pl.ds(off, CH)], iv)
        pltpu.sync_copy(ex_hbm.at[pl.ds(off, CH)], ebuf)
        cp1 = pltpu.async_copy(sa_hbm.at[iv], ra, sem1)
        cp2 = pltpu.async_copy(sb_hbm.at[iv], rb, sem2)
        cp1.wait()
        cp2.wait()

        def rows(r0, _):
            ri = (lane >> 3) + r0 * 2
            ev = plsc.load_gather(ebuf, [ri, ci])
            sav = plsc.load_gather(ra, [ri, ci])
            sbv = plsc.load_gather(rb, [ri, ci])
            ssum = sav + sbv + 1e-16
            mul = jnp.where(ci < 2, 1.0 / ssum, 1.0)
            plsc.store_scatter(obuf, [ri, ci], ev * mul)
            return 0

        lax.fori_loop(0, CH // 2, rows, 0, unroll=4)
        pltpu.sync_copy(obuf, out_hbm.at[pl.ds(off, CH)])
        return 0

    lax.fori_loop(0, (E // NW) // CH, chunk, 0)


def _edge_b_body(he_ref, att_ref, wm0_ref, wm1_ref, wv_ref, mix_ref, vr_ref):
    he = he_ref[...]
    att = att_ref[...]
    eb = he.shape[0]
    u0 = jnp.dot(he, wm0_ref[...], preferred_element_type=jnp.float32)
    u1 = jnp.dot(he, wm1_ref[...], preferred_element_type=jnp.float32)
    z = att[:, 0:1] * u0 + att[:, 1:2] * u1
    mix = jnp.tanh(z)
    mix_ref[...] = mix
    mv = jnp.dot(mix, wv_ref[...], preferred_element_type=jnp.float32)[:, :1]
    vr3 = mv * att[:, 2:5]
    vr_ref[...] = jnp.concatenate([vr3, jnp.zeros((eb, 5), jnp.float32)], axis=1)


def _edge_b(he, att, wm0, wm1, wv):
    eb = 2000
    return pl.pallas_call(
        _edge_b_body,
        grid=(E // eb,),
        in_specs=[
            pl.BlockSpec((eb, D), lambda i: (i, 0)),
            pl.BlockSpec((eb, 8), lambda i: (i, 0)),
            pl.BlockSpec((D, COEFF), lambda i: (0, 0)),
            pl.BlockSpec((D, COEFF), lambda i: (0, 0)),
            pl.BlockSpec((COEFF, 8), lambda i: (0, 0)),
        ],
        out_specs=[
            pl.BlockSpec((eb, COEFF), lambda i: (i, 0)),
            pl.BlockSpec((eb, 8), lambda i: (i, 0)),
        ],
        out_shape=[
            jax.ShapeDtypeStruct((E, COEFF), jnp.float32),
            jax.ShapeDtypeStruct((E, 8), jnp.float32),
        ],
    )(he, att, wm0, wm1, wv)


@functools.partial(
    pl.kernel,
    out_type=[jax.ShapeDtypeStruct((N, 64), jnp.float32) for _ in range(4)],
    scratch_types=[
        pltpu.VMEM((CH,), jnp.int32),
        pltpu.VMEM((CH, 64), jnp.float32),
        pltpu.VMEM((CH, 8), jnp.float32),
        pltpu.VMEM((CH, 64), jnp.float32),
        pltpu.VMEM_SHARED((N, 64), jnp.float32),
    ],
    **_SC_MESH,
)
def _scatter_hsem(he_hbm, att_hbm, idx_hbm, zeros_hbm, o0, o1, o2, o3,
                  iv, hbuf, abuf, pay, acc):
    c = lax.axis_index("c")
    s = lax.axis_index("s")
    base = s * (E // NS)
    outs = (o0, o1, o2, o3)

    for rr in range(2):
        @pl.when(s < 10)
        def _():
            pltpu.sync_copy(zeros_hbm.at[pl.ds(s * 1000, 1000), pl.ds(0, 64)],
                            acc.at[pl.ds(s * 1000, 1000)])

        plsc.subcore_barrier()

        def chunk(t, _):
            off = base + t * CH
            pltpu.sync_copy(idx_hbm.at[pl.ds(off, CH)], iv)
            pltpu.sync_copy(he_hbm.at[pl.ds(off, CH), pl.ds(rr * 64, 64)], hbuf)
            pltpu.sync_copy(att_hbm.at[pl.ds(off, CH)], abuf)

            def row(r, _):
                a = plsc.load_gather(abuf, [_splat(r), _splat(c)])
                for cg in range(4):
                    sl = pl.ds(cg * 16, 16)
                    pay[r, sl] = hbuf[r, sl] * a
                return 0

            lax.fori_loop(0, CH, row, 0, unroll=2)
            pltpu.sync_copy(pay, acc.at[iv], add=True)
            return 0

        lax.fori_loop(0, (E // NS) // CH, chunk, 0)
        plsc.subcore_barrier()

        for oi in range(4):
            @pl.when(jnp.logical_and(c * 2 + rr == oi, s < 10))
            def _(oref=outs[oi]):
                pltpu.sync_copy(acc.at[pl.ds(s * 1000, 1000)],
                                oref.at[pl.ds(s * 1000, 1000)])
        plsc.subcore_barrier()


@functools.partial(
    pl.kernel,
    out_type=[jax.ShapeDtypeStruct((N, 96), jnp.float32) for _ in range(8)],
    scratch_types=[
        pltpu.VMEM((CH,), jnp.int32),
        pltpu.VMEM((CH, 32), jnp.float32),
        pltpu.VMEM((CH, 8), jnp.float32),
        pltpu.VMEM((CH, 96), jnp.float32),
        pltpu.VMEM_SHARED((N, 96), jnp.float32),
    ],
    **_SC_MESH,
)
def _scatter_smat(mix_hbm, att_hbm, idx_hbm, zeros_hbm,
                  o0, o1, o2, o3, o4, o5, o6, o7,
                  iv, mbuf, abuf, pay, acc):
    c = lax.axis_index("c")
    s = lax.axis_index("s")
    base = s * (E // NS)
    outs = (o0, o1, o2, o3, o4, o5, o6, o7)

    for rr in range(4):
        @pl.when(s < 10)
        def _():
            pltpu.sync_copy(zeros_hbm.at[pl.ds(s * 1000, 1000), pl.ds(0, 96)],
                            acc.at[pl.ds(s * 1000, 1000)])

        plsc.subcore_barrier()
        col0 = (c * 4 + rr) * 32

        def chunk(t, _):
            off = base + t * CH
            pltpu.sync_copy(idx_hbm.at[pl.ds(off, CH)], iv)
            pltpu.sync_copy(mix_hbm.at[pl.ds(off, CH), pl.ds(col0, 32)], mbuf)
            pltpu.sync_copy(att_hbm.at[pl.ds(off, CH)], abuf)

            def row(r, _):
                rs = _splat(r)
                d0 = plsc.load_gather(abuf, [rs, _splat(2)])
                d1 = plsc.load_gather(abuf, [rs, _splat(3)])
                d2 = plsc.load_gather(abuf, [rs, _splat(4)])
                for cg in range(2):
                    m = mbuf[r, pl.ds(cg * 16, 16)]
                    pay[r, pl.ds(cg * 16, 16)] = m * d0
                    pay[r, pl.ds(32 + cg * 16, 16)] = m * d1
                    pay[r, pl.ds(64 + cg * 16, 16)] = m * d2
                return 0

            lax.fori_loop(0, CH, row, 0, unroll=2)
            pltpu.sync_copy(pay, acc.at[iv], add=True)
            return 0

        lax.fori_loop(0, (E // NS) // CH, chunk, 0)
        plsc.subcore_barrier()

        for oi in range(8):
            @pl.when(jnp.logical_and(c * 4 + rr == oi, s < 10))
            def _(oref=outs[oi]):
                pltpu.sync_copy(acc.at[pl.ds(s * 1000, 1000)],
                                oref.at[pl.ds(s * 1000, 1000)])
        plsc.subcore_barrier()


def _node_body(h_ref, x_ref, v_ref, sa_ref, sb_ref,
               hsa_ref, hsb_ref, hsc_ref, hsd_ref,
               s0_ref, s1_ref, s2_ref, s3_ref, s4_ref, s5_ref, s6_ref, s7_ref,
               va_ref, vb_ref,
               wp1_ref, bp1_ref, wp2_ref, bp2_ref,
               wn1h_ref, wn1s0_ref, wn1s1_ref, wn1sp_ref, bn1_ref,
               wn2_ref, bn2_ref, wv1_ref, bv1_ref, wv2_ref,
               hu_ref, xu_ref, vu_ref):
    h = h_ref[...]
    ssum = sa_ref[...] + sb_ref[...]
    cnt = ssum[:, 5:6]
    icnt = 1.0 / jnp.maximum(cnt, 1.0)
    schunks = [s0_ref[...], s1_ref[...], s2_ref[...], s3_ref[...],
               s4_ref[...], s5_ref[...], s6_ref[...], s7_ref[...]]
    norm_sq = jnp.zeros_like(h, shape=(h.shape[0], COEFF))
    for k in range(3):
        sk = jnp.concatenate([sc[:, 32 * k:32 * (k + 1)] for sc in schunks], axis=1)
        skm = sk * icnt
        norm_sq = norm_sq + skm * skm
    hsp = _silu(jnp.dot(norm_sq, wp1_ref[...], preferred_element_type=jnp.float32) + bp1_ref[...])
    hsp = _silu(jnp.dot(hsp, wp2_ref[...], preferred_element_type=jnp.float32) + bp2_ref[...])
    hs0 = jnp.concatenate([hsa_ref[...], hsb_ref[...]], axis=1)
    hs1 = jnp.concatenate([hsc_ref[...], hsd_ref[...]], axis=1)
    pre = (jnp.dot(h, wn1h_ref[...], preferred_element_type=jnp.float32)
           + jnp.dot(hs0, wn1s0_ref[...], preferred_element_type=jnp.float32)
           + jnp.dot(hs1, wn1s1_ref[...], preferred_element_type=jnp.float32)
           + jnp.dot(hsp, wn1sp_ref[...], preferred_element_type=jnp.float32)
           + bn1_ref[...])
    hu = h + _silu(jnp.dot(_silu(pre), wn2_ref[...], preferred_element_type=jnp.float32) + bn2_ref[...])
    hu_ref[...] = hu
    g = _silu(jnp.dot(h, wv1_ref[...], preferred_element_type=jnp.float32) + bv1_ref[...])
    gate = 2.0 / (1.0 + jnp.exp(-jnp.dot(g, wv2_ref[...], preferred_element_type=jnp.float32)[:, :1]))
    dv = (va_ref[...][:, :3] + vb_ref[...][:, :3]) * icnt
    vu = gate * v_ref[...] + dv
    vu_ref[...] = vu
    xu_ref[...] = x_ref[...] + vu


def _node_finalize(h, x, v, sa, sb, hs, schunks, va, vb, wp1, bp1, wp2,
                   bp2, wn1h, wn1s0, wn1s1, wn1sp, bn1, wn2, bn2, wv1, bv1, wv2):
    nb = 2000
    full = lambda r, c: pl.BlockSpec((r, c), lambda i: (0, 0))
    blk = lambda c: pl.BlockSpec((nb, c), lambda i: (i, 0))
    return pl.pallas_call(
        _node_body,
        grid=(N // nb,),
        in_specs=[
            blk(D), blk(3), blk(3), blk(8), blk(8),
            blk(64), blk(64), blk(64), blk(64),
            blk(96), blk(96), blk(96), blk(96),
            blk(96), blk(96), blk(96), blk(96),
            blk(8), blk(8),
            full(COEFF, D), full(1, D), full(D, D), full(1, D),
            full(D, D), full(D, D), full(D, D), full(D, D), full(1, D),
            full(D, D), full(1, D), full(D, D), full(1, D), full(D, 8),
        ],
        out_specs=[blk(D), blk(3), blk(3)],
        out_shape=[
            jax.ShapeDtypeStruct((N, D), jnp.float32),
            jax.ShapeDtypeStruct((N, 3), jnp.float32),
            jax.ShapeDtypeStruct((N, 3), jnp.float32),
        ],
    )(h, x, v, sa, sb, *hs, *schunks, va, vb, wp1, bp1, wp2, bp2,
      wn1h, wn1s0, wn1s1, wn1sp, bn1, wn2, bn2, wv1, bv1, wv2)


def kernel(h, x, v, pairlist, W_in, b_in, W_e1, b_e1, W_e2, b_e2, W_att, b_att,
           W_mix, W_v, W_n1, b_n1, W_n2, b_n2, W_p1, b_p1, W_p2, b_p2,
           W_vel1, b_vel1, W_vel2):
    f32 = jnp.float32
    idx_i = pairlist[0].astype(jnp.int32)
    idx_j = pairlist[1].astype(jnp.int32)

    z14 = jnp.zeros((D, PW - D - NRBF), f32)
    wi = jnp.concatenate([W_e1[:D], W_in[:D], z14], axis=1)
    wj = jnp.concatenate([W_e1[D:2 * D], W_in[D:], z14], axis=1)
    xpad = jnp.zeros((N, PW), f32).at[:, XC:XC + 3].set(x)
    xpi, xpj = -xpad, xpad
    we1f = W_e1[2 * D:2 * D + NRBF]
    wd = W_e1[2 * D + NRBF:2 * D + NRBF + 1]
    xpi = xpi.at[:, D:D + NRBF].add(0.5 * b_in[None, :])
    xpj = xpj.at[:, D:D + NRBF].add(0.5 * b_in[None, :])
    be1 = b_e1[None, :]
    be2 = b_e2[None, :]
    watt8 = jnp.zeros((D, 8), f32).at[:, :NH].set(W_att)
    batt8 = jnp.zeros((1, 8), f32).at[0, :NH].set(b_att)
    wm0 = W_mix[0::2]
    wm1 = W_mix[1::2]
    wv8 = jnp.zeros((COEFF, 8), f32).at[:, :1].set(W_v)
    wn1h = W_n1[:D]
    wn1sem = W_n1[D:D + COEFF].reshape(D, NH, D).transpose(1, 0, 2)
    wn1s0, wn1s1 = wn1sem[0], wn1sem[1]
    wn1sp = W_n1[D + COEFF:]
    wvel2_8 = jnp.zeros((D, 8), f32).at[:, :1].set(W_vel2)
    zeros_big = jnp.zeros((N, 192), f32)

    pi, pj = _nodeproj(h, xpi, xpj, wi, wj)
    elin = _edge_gather(pi, pj, idx_i, idx_j)
    he, ex = _edge_a(elin, we1f, wd, be1, W_e2, be2, watt8, batt8)
    sa, sb = _scatter_rows8(ex, idx_i, zeros_big)
    att = _edge_att(ex, sa, sb, idx_i)
    mix, vrow = _edge_b(he, att, wm0, wm1, wv8)
    hs = _scatter_hsem(he, att, idx_i, zeros_big)
    schunks = _scatter_smat(mix, att, idx_i, zeros_big)
    va, vb = _scatter_rows8(vrow, idx_i, zeros_big)
    hu, xu, vu = _node_finalize(
        h, x, v, sa, sb, list(hs), list(schunks), va, vb,
        W_p1, b_p1[None, :], W_p2, b_p2[None, :],
        wn1h, wn1s0, wn1s1, wn1sp, b_n1[None, :], W_n2, b_n2[None, :],
        W_vel1, b_vel1[None, :], wvel2_8)
    return (hu, xu, vu)


# speedup vs baseline: 8.8849x; 1.4133x over previous
import functools

import jax
import jax.numpy as jnp
from jax import lax
from jax.experimental import pallas as pl
from jax.experimental.pallas import tpu as pltpu
from jax.experimental.pallas import tpu_sc as plsc

N = 10000
E = 160000
D = 128
NRBF = 50
NH = 2
COEFF = 256
CUTOFF = 5.0
EPS = 1e-8
PW = 192
XC = D + NRBF

NC, NS = 2, 16
NW = NC * NS
CH = 200

_SC_MESH = dict(
    mesh=plsc.VectorSubcoreMesh(core_axis_name="c", subcore_axis_name="s"),
    compiler_params=pltpu.CompilerParams(use_tc_tiling_on_sc=False,
                                         needs_layout_passes=False),
)


def _splat(v):
    return jnp.broadcast_to(v, (16,)).astype(jnp.int32)


def _pipe_scatter_round(idx_hbm, data_hbm, dslice, att_hbm, acc, base, nch,
                        ivs, dbufs, abufs, pays, insems, scatsems, build):
    dc0, dw = dslice

    def issue_in(t, q, i4):
        off = base + t * CH
        pltpu.async_copy(idx_hbm.at[pl.ds(off, CH)], ivs[i4], insems[q])
        pltpu.async_copy(data_hbm.at[pl.ds(off, CH), pl.ds(dc0, dw)], dbufs[q],
                         insems[q])
        pltpu.async_copy(att_hbm.at[pl.ds(off, CH)], abufs[q], insems[q])

    def wait_in(p, i4):
        pltpu.make_async_copy(idx_hbm.at[pl.ds(0, CH)], ivs[i4], insems[p]).wait()
        pltpu.make_async_copy(data_hbm.at[pl.ds(0, CH), pl.ds(dc0, dw)],
                              dbufs[p], insems[p]).wait()
        pltpu.make_async_copy(att_hbm.at[pl.ds(0, CH)], abufs[p], insems[p]).wait()

    def wait_scat(p):
        pltpu.make_async_copy(pays[p], acc.at[ivs[0]], scatsems[p]).wait()

    def body(t, j, scat_wait, prefetch):
        p, i4 = j % 2, j % 4
        wait_in(p, i4)
        if scat_wait:
            wait_scat(p)
        if prefetch:
            issue_in(t + 1, 1 - p, (j + 1) % 4)
        build(p)
        pltpu.async_copy(pays[p], acc.at[ivs[i4]], scatsems[p], add=True)

    issue_in(0, 0, 0)
    body(0, 0, False, True)
    body(1, 1, False, True)
    body(2, 2, True, True)
    body(3, 3, True, True)

    def lbody(nt, _):
        t0 = 4 + nt * 4
        for j in range(4):
            body(t0 + j, j, True, True)
        return 0

    lax.fori_loop(0, (nch - 6) // 4, lbody, 0)
    body(nch - 2, (nch - 2) % 4, True, True)
    body(nch - 1, (nch - 1) % 4, True, False)
    wait_scat(0)
    wait_scat(1)


def _silu(z):
    return z * (1.0 / (1.0 + jnp.exp(-z)))


def _nodeproj_body(h_ref, xpi_ref, xpj_ref, wi_ref, wj_ref, pi_ref, pj_ref):
    h = h_ref[...]
    pi_ref[...] = jnp.dot(h, wi_ref[...], preferred_element_type=jnp.float32) + xpi_ref[...]
    pj_ref[...] = jnp.dot(h, wj_ref[...], preferred_element_type=jnp.float32) + xpj_ref[...]


def _nodeproj(h, xpi, xpj, wi, wj):
    nb = 2000
    return pl.pallas_call(
        _nodeproj_body,
        grid=(N // nb,),
        in_specs=[
            pl.BlockSpec((nb, D), lambda i: (i, 0)),
            pl.BlockSpec((nb, PW), lambda i: (i, 0)),
            pl.BlockSpec((nb, PW), lambda i: (i, 0)),
            pl.BlockSpec((D, PW), lambda i: (0, 0)),
            pl.BlockSpec((D, PW), lambda i: (0, 0)),
        ],
        out_specs=[
            pl.BlockSpec((nb, PW), lambda i: (i, 0)),
            pl.BlockSpec((nb, PW), lambda i: (i, 0)),
        ],
        out_shape=[
            jax.ShapeDtypeStruct((N, PW), jnp.float32),
            jax.ShapeDtypeStruct((N, PW), jnp.float32),
        ],
    )(h, xpi, xpj, wi, wj)


@functools.partial(
    pl.kernel,
    out_type=jax.ShapeDtypeStruct((E, PW), jnp.float32),
    scratch_types=[
        [pltpu.VMEM((CH,), jnp.int32) for _ in range(2)],
        [pltpu.VMEM((CH,), jnp.int32) for _ in range(2)],
        [pltpu.VMEM((CH, PW), jnp.float32) for _ in range(2)],
        [pltpu.SemaphoreType.DMA for _ in range(2)],
        [pltpu.SemaphoreType.DMA for _ in range(2)],
        [pltpu.SemaphoreType.DMA for _ in range(2)],
    ],
    **_SC_MESH,
)
def _edge_gather(pi_hbm, pj_hbm, ii_hbm, jj_hbm, out_hbm, ivs, jvs, bufs,
                 insems, gsems, outsems):
    wid = lax.axis_index("s") * NC + lax.axis_index("c")
    base = wid * (E // NW)
    nch = (E // NW) // CH

    def issue_in(t, q):
        off = base + t * CH
        pltpu.async_copy(ii_hbm.at[pl.ds(off, CH)], ivs[q], insems[q])
        pltpu.async_copy(jj_hbm.at[pl.ds(off, CH)], jvs[q], insems[q])

    def wait_in(p):
        pltpu.make_async_copy(ii_hbm.at[pl.ds(0, CH)], ivs[p], insems[p]).wait()
        pltpu.make_async_copy(jj_hbm.at[pl.ds(0, CH)], jvs[p], insems[p]).wait()

    def wait_out(p):
        pltpu.make_async_copy(bufs[p], out_hbm.at[pl.ds(0, CH)], outsems[p]).wait()

    def body(t, p, out_wait, prefetch):
        wait_in(p)
        if out_wait:
            wait_out(p)
        if prefetch:
            issue_in(t + 1, 1 - p)
        pltpu.async_copy(pi_hbm.at[ivs[p]], bufs[p], gsems[p]).wait()
        pltpu.async_copy(pj_hbm.at[jvs[p]], bufs[p], gsems[p], add=True).wait()
        pltpu.async_copy(bufs[p], out_hbm.at[pl.ds(base + t * CH, CH)], outsems[p])

    issue_in(0, 0)
    body(0, 0, False, True)
    body(1, 1, False, True)

    def lbody(nt, _):
        t0 = 2 + nt * 2
        body(t0, 0, True, True)
        body(t0 + 1, 1, True, True)
        return 0

    lax.fori_loop(0, (nch - 3) // 2, lbody, 0)
    body(nch - 1, (nch - 1) % 2, True, False)
    wait_out(0)
    wait_out(1)


def _edge_a_body(elin_ref, we1f_ref, wd_ref, be1_ref, we2_ref, be2_ref,
                 watt_ref, batt_ref, he_ref, ex_ref):
    elin = elin_ref[...]
    eb = elin.shape[0]
    bpart = elin[:, :D]
    apart = elin[:, D:D + NRBF]
    r3 = elin[:, XC:XC + 3]
    d2 = jnp.sum(r3 * r3, axis=1, keepdims=True) + 1e-24
    dd = jnp.sqrt(d2)
    dirv = r3 / (dd + EPS)
    step = CUTOFF / (NRBF - 1)
    centers = step * lax.broadcasted_iota(jnp.int32, (1, NRBF), 1).astype(jnp.float32)
    rbf = jnp.exp(-((dd - centers) ** 2) * (1.0 / (2.0 * step * step)))
    filt = rbf * apart
    e1 = (bpart + jnp.dot(filt, we1f_ref[...], preferred_element_type=jnp.float32)
          + dd * wd_ref[...] + be1_ref[...])
    he = jnp.dot(_silu(e1), we2_ref[...], preferred_element_type=jnp.float32) + be2_ref[...]
    he_ref[...] = he
    aw = jnp.dot(he, watt_ref[...], preferred_element_type=jnp.float32) + batt_ref[...]
    cw = jnp.where(aw > 0, aw, 2.0 * (jnp.exp(aw * 0.5) - 1.0))
    cc = jnp.where(dd < CUTOFF, 0.5 * (jnp.cos(jnp.pi / CUTOFF * dd) + 1.0), 0.0)
    ew = jnp.exp(cw * cc)[:, :2]
    ex_ref[...] = jnp.concatenate(
        [ew, dirv, jnp.ones((eb, 1), jnp.float32), jnp.zeros((eb, 2), jnp.float32)],
        axis=1)


def _edge_a(elin, we1f, wd, be1, we2, be2, watt, batt):
    eb = 2000
    return pl.pallas_call(
        _edge_a_body,
        grid=(E // eb,),
        in_specs=[
            pl.BlockSpec((eb, PW), lambda i: (i, 0)),
            pl.BlockSpec((NRBF, D), lambda i: (0, 0)),
            pl.BlockSpec((1, D), lambda i: (0, 0)),
            pl.BlockSpec((1, D), lambda i: (0, 0)),
            pl.BlockSpec((D, D), lambda i: (0, 0)),
            pl.BlockSpec((1, D), lambda i: (0, 0)),
            pl.BlockSpec((D, 8), lambda i: (0, 0)),
            pl.BlockSpec((1, 8), lambda i: (0, 0)),
        ],
        out_specs=[
            pl.BlockSpec((eb, D), lambda i: (i, 0)),
            pl.BlockSpec((eb, 8), lambda i: (i, 0)),
        ],
        out_shape=[
            jax.ShapeDtypeStruct((E, D), jnp.float32),
            jax.ShapeDtypeStruct((E, 8), jnp.float32),
        ],
    )(elin, we1f, wd, be1, we2, be2, watt, batt)


@functools.partial(
    pl.kernel,
    out_type=[
        jax.ShapeDtypeStruct((N, 8), jnp.float32),
        jax.ShapeDtypeStruct((N, 8), jnp.float32),
    ],
    scratch_types=[
        pltpu.VMEM((CH,), jnp.int32),
        pltpu.VMEM((CH, 8), jnp.float32),
        pltpu.VMEM_SHARED((N, 8), jnp.float32),
    ],
    **_SC_MESH,
)
def _scatter_rows8(rows_hbm, idx_hbm, zeros_hbm, out_a, out_b, iv, buf, acc):
    c = lax.axis_index("c")
    s = lax.axis_index("s")

    @pl.when(s < 10)
    def _():
        pltpu.sync_copy(zeros_hbm.at[pl.ds(s * 1000, 1000), pl.ds(0, 8)],
                        acc.at[pl.ds(s * 1000, 1000)])

    plsc.subcore_barrier()

    base = c * (E // NC) + s * (E // NC // NS)

    def chunk(t, _):
        off = base + t * CH
        pltpu.sync_copy(idx_hbm.at[pl.ds(off, CH)], iv)
        pltpu.sync_copy(rows_hbm.at[pl.ds(off, CH)], buf)
        pltpu.sync_copy(buf, acc.at[iv], add=True)
        return 0

    lax.fori_loop(0, (E // NC // NS) // CH, chunk, 0)
    plsc.subcore_barrier()

    @pl.when(jnp.logical_and(c == 0, s < 10))
    def _():
        pltpu.sync_copy(acc.at[pl.ds(s * 1000, 1000)],
                        out_a.at[pl.ds(s * 1000, 1000)])

    @pl.when(jnp.logical_and(c == 1, s < 10))
    def _():
        pltpu.sync_copy(acc.at[pl.ds(s * 1000, 1000)],
                        out_b.at[pl.ds(s * 1000, 1000)])


@functools.partial(
    pl.kernel,
    out_type=jax.ShapeDtypeStruct((E, 8), jnp.float32),
    scratch_types=[
        pltpu.VMEM((CH,), jnp.int32),
        pltpu.VMEM((CH, 8), jnp.float32),
        pltpu.VMEM((CH, 8), jnp.float32),
        pltpu.VMEM((CH, 8), jnp.float32),
        pltpu.VMEM((CH, 8), jnp.float32),
        pltpu.SemaphoreType.DMA,
        pltpu.SemaphoreType.DMA,
    ],
    **_SC_MESH,
)
def _edge_att(ex_hbm, sa_hbm, sb_hbm, idx_hbm, out_hbm, iv, ebuf, ra, rb, obuf,
              sem1, sem2):
    wid = lax.axis_index("s") * NC + lax.axis_index("c")
    base = wid * (E // NW)
    lane = lax.broadcasted_iota(jnp.int32, (16,), 0)
    ci = lane & 7

    def chunk(t, _):
        off = base + t * CH
        pltpu.sync_copy(idx_hbm.at[pl.ds(off, CH)], iv)
        pltpu.sync_copy(ex_hbm.at[pl.ds(off, CH)], ebuf)
        cp1 = pltpu.async_copy(sa_hbm.at[iv], ra, sem1)
        cp2 = pltpu.async_copy(sb_hbm.at[iv], rb, sem2)
        cp1.wait()
        cp2.wait()

        def rows(r0, _):
            ri = (lane >> 3) + r0 * 2
            ev = plsc.load_gather(ebuf, [ri, ci])
            sav = plsc.load_gather(ra, [ri, ci])
            sbv = plsc.load_gather(rb, [ri, ci])
            ssum = sav + sbv + 1e-16
            mul = jnp.where(ci < 2, 1.0 / ssum, 1.0)
            plsc.store_scatter(obuf, [ri, ci], ev * mul)
            return 0

        lax.fori_loop(0, CH // 2, rows, 0, unroll=4)
        pltpu.sync_copy(obuf, out_hbm.at[pl.ds(off, CH)])
        return 0

    lax.fori_loop(0, (E // NW) // CH, chunk, 0)


def _edge_b_body(he_ref, att_ref, wm0_ref, wm1_ref, wv_ref, mix_ref, vr_ref):
    he = he_ref[...]
    att = att_ref[...]
    eb = he.shape[0]
    u0 = jnp.dot(he, wm0_ref[...], preferred_element_type=jnp.float32)
    u1 = jnp.dot(he, wm1_ref[...], preferred_element_type=jnp.float32)
    z = att[:, 0:1] * u0 + att[:, 1:2] * u1
    mix = jnp.tanh(z)
    mix_ref[...] = mix
    mv = jnp.dot(mix, wv_ref[...], preferred_element_type=jnp.float32)[:, :1]
    vr3 = mv * att[:, 2:5]
    vr_ref[...] = jnp.concatenate([vr3, jnp.zeros((eb, 5), jnp.float32)], axis=1)


def _edge_b(he, att, wm0, wm1, wv):
    eb = 2000
    return pl.pallas_call(
        _edge_b_body,
        grid=(E // eb,),
        in_specs=[
            pl.BlockSpec((eb, D), lambda i: (i, 0)),
            pl.BlockSpec((eb, 8), lambda i: (i, 0)),
            pl.BlockSpec((D, COEFF), lambda i: (0, 0)),
            pl.BlockSpec((D, COEFF), lambda i: (0, 0)),
            pl.BlockSpec((COEFF, 8), lambda i: (0, 0)),
        ],
        out_specs=[
            pl.BlockSpec((eb, COEFF), lambda i: (i, 0)),
            pl.BlockSpec((eb, 8), lambda i: (i, 0)),
        ],
        out_shape=[
            jax.ShapeDtypeStruct((E, COEFF), jnp.float32),
            jax.ShapeDtypeStruct((E, 8), jnp.float32),
        ],
    )(he, att, wm0, wm1, wv)


@functools.partial(
    pl.kernel,
    out_type=[jax.ShapeDtypeStruct((N, 64), jnp.float32) for _ in range(4)],
    scratch_types=[
        [pltpu.VMEM((CH,), jnp.int32) for _ in range(4)],
        [pltpu.VMEM((CH, 64), jnp.float32) for _ in range(2)],
        [pltpu.VMEM((CH, 8), jnp.float32) for _ in range(2)],
        [pltpu.VMEM((CH, 64), jnp.float32) for _ in range(2)],
        [pltpu.SemaphoreType.DMA for _ in range(2)],
        [pltpu.SemaphoreType.DMA for _ in range(2)],
        pltpu.VMEM_SHARED((N, 64), jnp.float32),
    ],
    **_SC_MESH,
)
def _scatter_hsem(he_hbm, att_hbm, idx_hbm, zeros_hbm, o0, o1, o2, o3,
                  ivs, hbufs, abufs, pays, insems, scatsems, acc):
    c = lax.axis_index("c")
    s = lax.axis_index("s")
    base = s * (E // NS)
    outs = (o0, o1, o2, o3)

    for rr in range(2):
        @pl.when(s < 10)
        def _():
            pltpu.sync_copy(zeros_hbm.at[pl.ds(s * 1000, 1000), pl.ds(0, 64)],
                            acc.at[pl.ds(s * 1000, 1000)])

        plsc.subcore_barrier()

        def build(p):
            hbuf, abuf, pay = hbufs[p], abufs[p], pays[p]

            def row(r, _):
                a = plsc.load_gather(abuf, [_splat(r), _splat(c)])
                for cg in range(4):
                    sl = pl.ds(cg * 16, 16)
                    pay[r, sl] = hbuf[r, sl] * a
                return 0

            lax.fori_loop(0, CH, row, 0, unroll=4)

        _pipe_scatter_round(idx_hbm, he_hbm, (rr * 64, 64), att_hbm, acc,
                            base, (E // NS) // CH, ivs, hbufs, abufs, pays,
                            insems, scatsems, build)
        plsc.subcore_barrier()

        for oi in range(4):
            @pl.when(jnp.logical_and(c * 2 + rr == oi, s < 10))
            def _(oref=outs[oi]):
                pltpu.sync_copy(acc.at[pl.ds(s * 1000, 1000)],
                                oref.at[pl.ds(s * 1000, 1000)])
        plsc.subcore_barrier()


@functools.partial(
    pl.kernel,
    out_type=[jax.ShapeDtypeStruct((N, 96), jnp.float32) for _ in range(8)],
    scratch_types=[
        [pltpu.VMEM((CH,), jnp.int32) for _ in range(4)],
        [pltpu.VMEM((CH, 32), jnp.float32) for _ in range(2)],
        [pltpu.VMEM((CH, 8), jnp.float32) for _ in range(2)],
        [pltpu.VMEM((CH, 96), jnp.float32) for _ in range(2)],
        [pltpu.SemaphoreType.DMA for _ in range(2)],
        [pltpu.SemaphoreType.DMA for _ in range(2)],
        pltpu.VMEM_SHARED((N, 96), jnp.float32),
    ],
    **_SC_MESH,
)
def _scatter_smat(mix_hbm, att_hbm, idx_hbm, zeros_hbm,
                  o0, o1, o2, o3, o4, o5, o6, o7,
                  ivs, mbufs, abufs, pays, insems, scatsems, acc):
    c = lax.axis_index("c")
    s = lax.axis_index("s")
    base = s * (E // NS)
    outs = (o0, o1, o2, o3, o4, o5, o6, o7)

    for rr in range(4):
        @pl.when(s < 10)
        def _():
            pltpu.sync_copy(zeros_hbm.at[pl.ds(s * 1000, 1000), pl.ds(0, 96)],
                            acc.at[pl.ds(s * 1000, 1000)])

        plsc.subcore_barrier()
        col0 = (c * 4 + rr) * 32

        def build(p):
            mbuf, abuf, pay = mbufs[p], abufs[p], pays[p]

            def row(r, _):
                rs = _splat(r)
                d0 = plsc.load_gather(abuf, [rs, _splat(2)])
                d1 = plsc.load_gather(abuf, [rs, _splat(3)])
                d2 = plsc.load_gather(abuf, [rs, _splat(4)])
                for cg in range(2):
                    m = mbuf[r, pl.ds(cg * 16, 16)]
                    pay[r, pl.ds(cg * 16, 16)] = m * d0
                    pay[r, pl.ds(32 + cg * 16, 16)] = m * d1
                    pay[r, pl.ds(64 + cg * 16, 16)] = m * d2
                return 0

            lax.fori_loop(0, CH, row, 0, unroll=4)

        _pipe_scatter_round(idx_hbm, mix_hbm, (col0, 32), att_hbm, acc,
                            base, (E // NS) // CH, ivs, mbufs, abufs, pays,
                            insems, scatsems, build)
        plsc.subcore_barrier()

        for oi in range(8):
            @pl.when(jnp.logical_and(c * 4 + rr == oi, s < 10))
            def _(oref=outs[oi]):
                pltpu.sync_copy(acc.at[pl.ds(s * 1000, 1000)],
                                oref.at[pl.ds(s * 1000, 1000)])
        plsc.subcore_barrier()


def _node_body(h_ref, x_ref, v_ref, sa_ref, sb_ref,
               hsa_ref, hsb_ref, hsc_ref, hsd_ref,
               s0_ref, s1_ref, s2_ref, s3_ref, s4_ref, s5_ref, s6_ref, s7_ref,
               va_ref, vb_ref,
               wp1_ref, bp1_ref, wp2_ref, bp2_ref,
               wn1h_ref, wn1s0_ref, wn1s1_ref, wn1sp_ref, bn1_ref,
               wn2_ref, bn2_ref, wv1_ref, bv1_ref, wv2_ref,
               hu_ref, xu_ref, vu_ref):
    h = h_ref[...]
    ssum = sa_ref[...] + sb_ref[...]
    cnt = ssum[:, 5:6]
    icnt = 1.0 / jnp.maximum(cnt, 1.0)
    schunks = [s0_ref[...], s1_ref[...], s2_ref[...], s3_ref[...],
               s4_ref[...], s5_ref[...], s6_ref[...], s7_ref[...]]
    norm_sq = jnp.zeros_like(h, shape=(h.shape[0], COEFF))
    for k in range(3):
        sk = jnp.concatenate([sc[:, 32 * k:32 * (k + 1)] for sc in schunks], axis=1)
        skm = sk * icnt
        norm_sq = norm_sq + skm * skm
    hsp = _silu(jnp.dot(norm_sq, wp1_ref[...], preferred_element_type=jnp.float32) + bp1_ref[...])
    hsp = _silu(jnp.dot(hsp, wp2_ref[...], preferred_element_type=jnp.float32) + bp2_ref[...])
    hs0 = jnp.concatenate([hsa_ref[...], hsb_ref[...]], axis=1)
    hs1 = jnp.concatenate([hsc_ref[...], hsd_ref[...]], axis=1)
    pre = (jnp.dot(h, wn1h_ref[...], preferred_element_type=jnp.float32)
           + jnp.dot(hs0, wn1s0_ref[...], preferred_element_type=jnp.float32)
           + jnp.dot(hs1, wn1s1_ref[...], preferred_element_type=jnp.float32)
           + jnp.dot(hsp, wn1sp_ref[...], preferred_element_type=jnp.float32)
           + bn1_ref[...])
    hu = h + _silu(jnp.dot(_silu(pre), wn2_ref[...], preferred_element_type=jnp.float32) + bn2_ref[...])
    hu_ref[...] = hu
    g = _silu(jnp.dot(h, wv1_ref[...], preferred_element_type=jnp.float32) + bv1_ref[...])
    gate = 2.0 / (1.0 + jnp.exp(-jnp.dot(g, wv2_ref[...], preferred_element_type=jnp.float32)[:, :1]))
    dv = (va_ref[...][:, :3] + vb_ref[...][:, :3]) * icnt
    vu = gate * v_ref[...] + dv
    vu_ref[...] = vu
    xu_ref[...] = x_ref[...] + vu


def _node_finalize(h, x, v, sa, sb, hs, schunks, va, vb, wp1, bp1, wp2,
                   bp2, wn1h, wn1s0, wn1s1, wn1sp, bn1, wn2, bn2, wv1, bv1, wv2):
    nb = 2000
    full = lambda r, c: pl.BlockSpec((r, c), lambda i: (0, 0))
    blk = lambda c: pl.BlockSpec((nb, c), lambda i: (i, 0))
    return pl.pallas_call(
        _node_body,
        grid=(N // nb,),
        in_specs=[
            blk(D), blk(3), blk(3), blk(8), blk(8),
            blk(64), blk(64), blk(64), blk(64),
            blk(96), blk(96), blk(96), blk(96),
            blk(96), blk(96), blk(96), blk(96),
            blk(8), blk(8),
            full(COEFF, D), full(1, D), full(D, D), full(1, D),
            full(D, D), full(D, D), full(D, D), full(D, D), full(1, D),
            full(D, D), full(1, D), full(D, D), full(1, D), full(D, 8),
        ],
        out_specs=[blk(D), blk(3), blk(3)],
        out_shape=[
            jax.ShapeDtypeStruct((N, D), jnp.float32),
            jax.ShapeDtypeStruct((N, 3), jnp.float32),
            jax.ShapeDtypeStruct((N, 3), jnp.float32),
        ],
    )(h, x, v, sa, sb, *hs, *schunks, va, vb, wp1, bp1, wp2, bp2,
      wn1h, wn1s0, wn1s1, wn1sp, bn1, wn2, bn2, wv1, bv1, wv2)


def kernel(h, x, v, pairlist, W_in, b_in, W_e1, b_e1, W_e2, b_e2, W_att, b_att,
           W_mix, W_v, W_n1, b_n1, W_n2, b_n2, W_p1, b_p1, W_p2, b_p2,
           W_vel1, b_vel1, W_vel2):
    f32 = jnp.float32
    idx_i = pairlist[0].astype(jnp.int32)
    idx_j = pairlist[1].astype(jnp.int32)

    z14 = jnp.zeros((D, PW - D - NRBF), f32)
    wi = jnp.concatenate([W_e1[:D], W_in[:D], z14], axis=1)
    wj = jnp.concatenate([W_e1[D:2 * D], W_in[D:], z14], axis=1)
    xpad = jnp.zeros((N, PW), f32).at[:, XC:XC + 3].set(x)
    xpi, xpj = -xpad, xpad
    we1f = W_e1[2 * D:2 * D + NRBF]
    wd = W_e1[2 * D + NRBF:2 * D + NRBF + 1]
    xpi = xpi.at[:, D:D + NRBF].add(0.5 * b_in[None, :])
    xpj = xpj.at[:, D:D + NRBF].add(0.5 * b_in[None, :])
    be1 = b_e1[None, :]
    be2 = b_e2[None, :]
    watt8 = jnp.zeros((D, 8), f32).at[:, :NH].set(W_att)
    batt8 = jnp.zeros((1, 8), f32).at[0, :NH].set(b_att)
    wm0 = W_mix[0::2]
    wm1 = W_mix[1::2]
    wv8 = jnp.zeros((COEFF, 8), f32).at[:, :1].set(W_v)
    wn1h = W_n1[:D]
    wn1sem = W_n1[D:D + COEFF].reshape(D, NH, D).transpose(1, 0, 2)
    wn1s0, wn1s1 = wn1sem[0], wn1sem[1]
    wn1sp = W_n1[D + COEFF:]
    wvel2_8 = jnp.zeros((D, 8), f32).at[:, :1].set(W_vel2)
    zeros_big = jnp.zeros((N, 192), f32)

    pi, pj = _nodeproj(h, xpi, xpj, wi, wj)
    elin = _edge_gather(pi, pj, idx_i, idx_j)
    he, ex = _edge_a(elin, we1f, wd, be1, W_e2, be2, watt8, batt8)
    sa, sb = _scatter_rows8(ex, idx_i, zeros_big)
    att = _edge_att(ex, sa, sb, idx_i)
    mix, vrow = _edge_b(he, att, wm0, wm1, wv8)
    hs = _scatter_hsem(he, att, idx_i, zeros_big)
    schunks = _scatter_smat(mix, att, idx_i, zeros_big)
    va, vb = _scatter_rows8(vrow, idx_i, zeros_big)
    hu, xu, vu = _node_finalize(
        h, x, v, sa, sb, list(hs), list(schunks), va, vb,
        W_p1, b_p1[None, :], W_p2, b_p2[None, :],
        wn1h, wn1s0, wn1s1, wn1sp, b_n1[None, :], W_n2, b_n2[None, :],
        W_vel1, b_vel1[None, :], wvel2_8)
    return (hu, xu, vu)


# speedup vs baseline: 10.5188x; 1.1839x over previous
import functools

import jax
import jax.numpy as jnp
from jax import lax
from jax.experimental import pallas as pl
from jax.experimental.pallas import tpu as pltpu
from jax.experimental.pallas import tpu_sc as plsc

N = 10000
E = 160000
D = 128
NRBF = 50
NH = 2
COEFF = 256
CUTOFF = 5.0
EPS = 1e-8
PW = 256
XC = D + NRBF

NC, NS = 2, 16
NW = NC * NS
CH = 200

_SC_MESH = dict(
    mesh=plsc.VectorSubcoreMesh(core_axis_name="c", subcore_axis_name="s"),
    compiler_params=pltpu.CompilerParams(use_tc_tiling_on_sc=False,
                                         needs_layout_passes=False),
)


def _splat(v):
    return jnp.broadcast_to(v, (16,)).astype(jnp.int32)


def _pipe_scatter_round(idx_hbm, data_hbm, dslice, att_hbm, acc, base, nch,
                        ivs, dbufs, abufs, pays, insems, scatsems, build):
    dc0, dw = dslice

    def issue_in(t, q, i4):
        off = base + t * CH
        pltpu.async_copy(idx_hbm.at[pl.ds(off, CH)], ivs[i4], insems[q])
        pltpu.async_copy(data_hbm.at[pl.ds(off, CH), pl.ds(dc0, dw)], dbufs[q],
                         insems[q])
        pltpu.async_copy(att_hbm.at[pl.ds(off, CH)], abufs[q], insems[q])

    def wait_in(p, i4):
        pltpu.make_async_copy(idx_hbm.at[pl.ds(0, CH)], ivs[i4], insems[p]).wait()
        pltpu.make_async_copy(data_hbm.at[pl.ds(0, CH), pl.ds(dc0, dw)],
                              dbufs[p], insems[p]).wait()
        pltpu.make_async_copy(att_hbm.at[pl.ds(0, CH)], abufs[p], insems[p]).wait()

    def wait_scat(p):
        pltpu.make_async_copy(pays[p], acc.at[ivs[0]], scatsems[p]).wait()

    def body(t, j, scat_wait, prefetch):
        p, i4 = j % 2, j % 4
        wait_in(p, i4)
        if scat_wait:
            wait_scat(p)
        if prefetch:
            issue_in(t + 1, 1 - p, (j + 1) % 4)
        build(p)
        pltpu.async_copy(pays[p], acc.at[ivs[i4]], scatsems[p], add=True)

    issue_in(0, 0, 0)
    body(0, 0, False, True)
    body(1, 1, False, True)
    body(2, 2, True, True)
    body(3, 3, True, True)

    def lbody(nt, _):
        t0 = 4 + nt * 4
        for j in range(4):
            body(t0 + j, j, True, True)
        return 0

    lax.fori_loop(0, (nch - 6) // 4, lbody, 0)
    body(nch - 2, (nch - 2) % 4, True, True)
    body(nch - 1, (nch - 1) % 4, True, False)
    wait_scat(0)
    wait_scat(1)


def _silu(z):
    return z * (1.0 / (1.0 + jnp.exp(-z)))


def _nodeproj_body(h_ref, xpi_ref, xpj_ref, wi_ref, wj_ref, pi_ref, pj_ref):
    h = h_ref[...]
    pi_ref[...] = jnp.dot(h, wi_ref[...], preferred_element_type=jnp.float32) + xpi_ref[...]
    pj_ref[...] = jnp.dot(h, wj_ref[...], preferred_element_type=jnp.float32) + xpj_ref[...]


def _nodeproj(h, xpi, xpj, wi, wj):
    nb = 2000
    return pl.pallas_call(
        _nodeproj_body,
        grid=(N // nb,),
        in_specs=[
            pl.BlockSpec((nb, D), lambda i: (i, 0)),
            pl.BlockSpec((nb, PW), lambda i: (i, 0)),
            pl.BlockSpec((nb, PW), lambda i: (i, 0)),
            pl.BlockSpec((D, PW), lambda i: (0, 0)),
            pl.BlockSpec((D, PW), lambda i: (0, 0)),
        ],
        out_specs=[
            pl.BlockSpec((nb, PW), lambda i: (i, 0)),
            pl.BlockSpec((nb, PW), lambda i: (i, 0)),
        ],
        out_shape=[
            jax.ShapeDtypeStruct((N, PW), jnp.float32),
            jax.ShapeDtypeStruct((N, PW), jnp.float32),
        ],
    )(h, xpi, xpj, wi, wj)


@functools.partial(
    pl.kernel,
    out_type=jax.ShapeDtypeStruct((E, PW), jnp.float32),
    scratch_types=[
        [pltpu.VMEM((CH,), jnp.int32) for _ in range(2)],
        [pltpu.VMEM((CH,), jnp.int32) for _ in range(2)],
        [pltpu.VMEM((CH, PW), jnp.float32) for _ in range(2)],
        [pltpu.SemaphoreType.DMA for _ in range(2)],
        [pltpu.SemaphoreType.DMA for _ in range(2)],
        [pltpu.SemaphoreType.DMA for _ in range(2)],
    ],
    mesh=plsc.VectorSubcoreMesh(core_axis_name="c", subcore_axis_name="s"),
    compiler_params=pltpu.CompilerParams(use_tc_tiling_on_sc=True),
)
def _edge_gather(pi_hbm, pj_hbm, ii_hbm, jj_hbm, out_hbm, ivs, jvs, bufs,
                 insems, gsems, outsems):
    wid = lax.axis_index("s") * NC + lax.axis_index("c")
    base = wid * (E // NW)
    nch = (E // NW) // CH

    def issue_in(t, q):
        off = base + t * CH
        pltpu.async_copy(ii_hbm.at[pl.ds(off, CH)], ivs[q], insems[q])
        pltpu.async_copy(jj_hbm.at[pl.ds(off, CH)], jvs[q], insems[q])

    def wait_in(p):
        pltpu.make_async_copy(ii_hbm.at[pl.ds(0, CH)], ivs[p], insems[p]).wait()
        pltpu.make_async_copy(jj_hbm.at[pl.ds(0, CH)], jvs[p], insems[p]).wait()

    def wait_out(p):
        pltpu.make_async_copy(bufs[p], out_hbm.at[pl.ds(0, CH)], outsems[p]).wait()

    def body(t, p, out_wait, prefetch):
        wait_in(p)
        if out_wait:
            wait_out(p)
        if prefetch:
            issue_in(t + 1, 1 - p)
        pltpu.async_copy(pi_hbm.at[ivs[p]], bufs[p], gsems[p]).wait()
        pltpu.async_copy(pj_hbm.at[jvs[p]], bufs[p], gsems[p], add=True).wait()
        pltpu.async_copy(bufs[p], out_hbm.at[pl.ds(base + t * CH, CH)], outsems[p])

    issue_in(0, 0)
    body(0, 0, False, True)
    body(1, 1, False, True)

    def lbody(nt, _):
        t0 = 2 + nt * 2
        body(t0, 0, True, True)
        body(t0 + 1, 1, True, True)
        return 0

    lax.fori_loop(0, (nch - 3) // 2, lbody, 0)
    body(nch - 1, (nch - 1) % 2, True, False)
    wait_out(0)
    wait_out(1)


def _edge_a_body(elin_ref, we1f_ref, wd_ref, be1_ref, we2_ref, be2_ref,
                 watt_ref, batt_ref, he_ref, ex_ref):
    elin = elin_ref[...]
    eb = elin.shape[0]
    bpart = elin[:, :D]
    g2 = elin[:, D:]
    lane = lax.broadcasted_iota(jnp.int32, (1, PW - D), 1)
    am = lane < NRBF
    xm = jnp.logical_and(lane >= NRBF, lane < NRBF + 3)
    xv = jnp.where(xm, g2, 0.0)
    d2 = jnp.sum(xv * xv, axis=1, keepdims=True) + 1e-24
    dd = jnp.sqrt(d2)
    dirg = xv / (dd + EPS)
    step = CUTOFF / (NRBF - 1)
    cl = step * lane.astype(jnp.float32)
    filt = jnp.exp(-((dd - cl) ** 2) * (1.0 / (2.0 * step * step))) * g2
    e1 = (bpart + jnp.dot(filt, we1f_ref[...], preferred_element_type=jnp.float32)
          + dd * wd_ref[...] + be1_ref[...])
    he = jnp.dot(_silu(e1), we2_ref[...], preferred_element_type=jnp.float32) + be2_ref[...]
    he_ref[...] = he
    aw = jnp.dot(he, watt_ref[...], preferred_element_type=jnp.float32) + batt_ref[...]
    cw = jnp.where(aw > 0, aw, 2.0 * (jnp.exp(aw * 0.5) - 1.0))
    z = d2 * (1.0 / (CUTOFF * CUTOFF))
    CPOLY = (0.9999999961449239, -2.467400694185487, 2.0293491311347505,
             -0.6675872267059777, 0.11753168588253822, -0.012695555692821478,
             0.0008026813884714353)
    pz = CPOLY[6]
    for k in (5, 4, 3, 2, 1, 0):
        pz = pz * z + CPOLY[k]
    cc = jnp.where(z < 1.0, pz, 0.0)
    ew = jnp.exp(cw * cc)[:, :2]
    ex_ref[...] = jnp.concatenate(
        [ew, dirg[:, NRBF:NRBF + 3], jnp.ones((eb, 1), jnp.float32),
         jnp.zeros((eb, 2), jnp.float32)], axis=1)


def _edge_a(elin, we1f, wd, be1, we2, be2, watt, batt):
    eb = 2000
    return pl.pallas_call(
        _edge_a_body,
        grid=(E // eb,),
        in_specs=[
            pl.BlockSpec((eb, PW), lambda i: (i, 0)),
            pl.BlockSpec((PW - D, D), lambda i: (0, 0)),
            pl.BlockSpec((1, D), lambda i: (0, 0)),
            pl.BlockSpec((1, D), lambda i: (0, 0)),
            pl.BlockSpec((D, D), lambda i: (0, 0)),
            pl.BlockSpec((1, D), lambda i: (0, 0)),
            pl.BlockSpec((D, 8), lambda i: (0, 0)),
            pl.BlockSpec((1, 8), lambda i: (0, 0)),
        ],
        out_specs=[
            pl.BlockSpec((eb, D), lambda i: (i, 0)),
            pl.BlockSpec((eb, 8), lambda i: (i, 0)),
        ],
        out_shape=[
            jax.ShapeDtypeStruct((E, D), jnp.float32),
            jax.ShapeDtypeStruct((E, 8), jnp.float32),
        ],
    )(elin, we1f, wd, be1, we2, be2, watt, batt)


@functools.partial(
    pl.kernel,
    out_type=[
        jax.ShapeDtypeStruct((N, 8), jnp.float32),
        jax.ShapeDtypeStruct((N, 8), jnp.float32),
    ],
    scratch_types=[
        pltpu.VMEM((CH,), jnp.int32),
        pltpu.VMEM((CH, 8), jnp.float32),
        pltpu.VMEM_SHARED((N, 8), jnp.float32),
    ],
    **_SC_MESH,
)
def _scatter_rows8(rows_hbm, idx_hbm, zeros_hbm, out_a, out_b, iv, buf, acc):
    c = lax.axis_index("c")
    s = lax.axis_index("s")

    @pl.when(s < 10)
    def _():
        pltpu.sync_copy(zeros_hbm.at[pl.ds(s * 1000, 1000), pl.ds(0, 8)],
                        acc.at[pl.ds(s * 1000, 1000)])

    plsc.subcore_barrier()

    base = c * (E // NC) + s * (E // NC // NS)

    def chunk(t, _):
        off = base + t * CH
        pltpu.sync_copy(idx_hbm.at[pl.ds(off, CH)], iv)
        pltpu.sync_copy(rows_hbm.at[pl.ds(off, CH)], buf)
        pltpu.sync_copy(buf, acc.at[iv], add=True)
        return 0

    lax.fori_loop(0, (E // NC // NS) // CH, chunk, 0)
    plsc.subcore_barrier()

    @pl.when(jnp.logical_and(c == 0, s < 10))
    def _():
        pltpu.sync_copy(acc.at[pl.ds(s * 1000, 1000)],
                        out_a.at[pl.ds(s * 1000, 1000)])

    @pl.when(jnp.logical_and(c == 1, s < 10))
    def _():
        pltpu.sync_copy(acc.at[pl.ds(s * 1000, 1000)],
                        out_b.at[pl.ds(s * 1000, 1000)])


@functools.partial(
    pl.kernel,
    out_type=jax.ShapeDtypeStruct((E, 8), jnp.float32),
    scratch_types=[
        pltpu.VMEM((CH,), jnp.int32),
        pltpu.VMEM((CH, 8), jnp.float32),
        pltpu.VMEM((CH, 8), jnp.float32),
        pltpu.VMEM((CH, 8), jnp.float32),
        pltpu.VMEM((CH, 8), jnp.float32),
        pltpu.SemaphoreType.DMA,
        pltpu.SemaphoreType.DMA,
    ],
    **_SC_MESH,
)
def _edge_att(ex_hbm, sa_hbm, sb_hbm, idx_hbm, out_hbm, iv, ebuf, ra, rb, obuf,
              sem1, sem2):
    wid = lax.axis_index("s") * NC + lax.axis_index("c")
    base = wid * (E // NW)
    lane = lax.broadcasted_iota(jnp.int32, (16,), 0)
    ci = lane & 7

    def chunk(t, _):
        off = base + t * CH
        pltpu.sync_copy(idx_hbm.at[pl.ds(off, CH)], iv)
        pltpu.sync_copy(ex_hbm.at[pl.ds(off, CH)], ebuf)
        cp1 = pltpu.async_copy(sa_hbm.at[iv], ra, sem1)
        cp2 = pltpu.async_copy(sb_hbm.at[iv], rb, sem2)
        cp1.wait()
        cp2.wait()

        def rows(r0, _):
            ri = (lane >> 3) + r0 * 2
            ev = plsc.load_gather(ebuf, [ri, ci])
            sav = plsc.load_gather(ra, [ri, ci])
            sbv = plsc.load_gather(rb, [ri, ci])
            ssum = sav + sbv + 1e-16
            mul = jnp.where(ci < 2, 1.0 / ssum, 1.0)
            plsc.store_scatter(obuf, [ri, ci], ev * mul)
            return 0

        lax.fori_loop(0, CH // 2, rows, 0, unroll=4)
        pltpu.sync_copy(obuf, out_hbm.at[pl.ds(off, CH)])
        return 0

    lax.fori_loop(0, (E // NW) // CH, chunk, 0)


def _edge_b_body(he_ref, att_ref, wm0_ref, wm1_ref, wv_ref, mix_ref, vr_ref):
    he = he_ref[...]
    att = att_ref[...]
    eb = he.shape[0]
    u0 = jnp.dot(he, wm0_ref[...], preferred_element_type=jnp.float32)
    u1 = jnp.dot(he, wm1_ref[...], preferred_element_type=jnp.float32)
    z = att[:, 0:1] * u0 + att[:, 1:2] * u1
    mix = jnp.tanh(z)
    mix_ref[...] = mix
    mv = jnp.dot(mix, wv_ref[...], preferred_element_type=jnp.float32)[:, :1]
    vr3 = mv * att[:, 2:5]
    vr_ref[...] = jnp.concatenate([vr3, jnp.zeros((eb, 5), jnp.float32)], axis=1)


def _edge_b(he, att, wm0, wm1, wv):
    eb = 2000
    return pl.pallas_call(
        _edge_b_body,
        grid=(E // eb,),
        in_specs=[
            pl.BlockSpec((eb, D), lambda i: (i, 0)),
            pl.BlockSpec((eb, 8), lambda i: (i, 0)),
            pl.BlockSpec((D, COEFF), lambda i: (0, 0)),
            pl.BlockSpec((D, COEFF), lambda i: (0, 0)),
            pl.BlockSpec((COEFF, 8), lambda i: (0, 0)),
        ],
        out_specs=[
            pl.BlockSpec((eb, COEFF), lambda i: (i, 0)),
            pl.BlockSpec((eb, 8), lambda i: (i, 0)),
        ],
        out_shape=[
            jax.ShapeDtypeStruct((E, COEFF), jnp.float32),
            jax.ShapeDtypeStruct((E, 8), jnp.float32),
        ],
    )(he, att, wm0, wm1, wv)


@functools.partial(
    pl.kernel,
    out_type=[jax.ShapeDtypeStruct((N, 64), jnp.float32) for _ in range(4)],
    scratch_types=[
        [pltpu.VMEM((CH,), jnp.int32) for _ in range(4)],
        [pltpu.VMEM((CH, 64), jnp.float32) for _ in range(2)],
        [pltpu.VMEM((CH, 8), jnp.float32) for _ in range(2)],
        [pltpu.VMEM((CH, 64), jnp.float32) for _ in range(2)],
        [pltpu.SemaphoreType.DMA for _ in range(2)],
        [pltpu.SemaphoreType.DMA for _ in range(2)],
        pltpu.VMEM_SHARED((N, 64), jnp.float32),
    ],
    **_SC_MESH,
)
def _scatter_hsem(he_hbm, att_hbm, idx_hbm, zeros_hbm, o0, o1, o2, o3,
                  ivs, hbufs, abufs, pays, insems, scatsems, acc):
    c = lax.axis_index("c")
    s = lax.axis_index("s")
    base = s * (E // NS)
    outs = (o0, o1, o2, o3)

    for rr in range(2):
        @pl.when(s < 10)
        def _():
            pltpu.sync_copy(zeros_hbm.at[pl.ds(s * 1000, 1000), pl.ds(0, 64)],
                            acc.at[pl.ds(s * 1000, 1000)])

        plsc.subcore_barrier()

        def build(p):
            hbuf, abuf, pay = hbufs[p], abufs[p], pays[p]

            def row(r, _):
                a = plsc.load_gather(abuf, [_splat(r), _splat(c)])
                for cg in range(4):
                    sl = pl.ds(cg * 16, 16)
                    pay[r, sl] = hbuf[r, sl] * a
                return 0

            lax.fori_loop(0, CH, row, 0, unroll=4)

        _pipe_scatter_round(idx_hbm, he_hbm, (rr * 64, 64), att_hbm, acc,
                            base, (E // NS) // CH, ivs, hbufs, abufs, pays,
                            insems, scatsems, build)
        plsc.subcore_barrier()

        for oi in range(4):
            @pl.when(jnp.logical_and(c * 2 + rr == oi, s < 10))
            def _(oref=outs[oi]):
                pltpu.sync_copy(acc.at[pl.ds(s * 1000, 1000)],
                                oref.at[pl.ds(s * 1000, 1000)])
        plsc.subcore_barrier()


@functools.partial(
    pl.kernel,
    out_type=[jax.ShapeDtypeStruct((N, 96), jnp.float32) for _ in range(8)],
    scratch_types=[
        [pltpu.VMEM((CH,), jnp.int32) for _ in range(4)],
        [pltpu.VMEM((CH, 32), jnp.float32) for _ in range(2)],
        [pltpu.VMEM((CH, 8), jnp.float32) for _ in range(2)],
        [pltpu.VMEM((CH, 96), jnp.float32) for _ in range(2)],
        [pltpu.SemaphoreType.DMA for _ in range(2)],
        [pltpu.SemaphoreType.DMA for _ in range(2)],
        pltpu.VMEM_SHARED((N, 96), jnp.float32),
    ],
    **_SC_MESH,
)
def _scatter_smat(mix_hbm, att_hbm, idx_hbm, zeros_hbm,
                  o0, o1, o2, o3, o4, o5, o6, o7,
                  ivs, mbufs, abufs, pays, insems, scatsems, acc):
    c = lax.axis_index("c")
    s = lax.axis_index("s")
    base = s * (E // NS)
    outs = (o0, o1, o2, o3, o4, o5, o6, o7)

    for rr in range(4):
        @pl.when(s < 10)
        def _():
            pltpu.sync_copy(zeros_hbm.at[pl.ds(s * 1000, 1000), pl.ds(0, 96)],
                            acc.at[pl.ds(s * 1000, 1000)])

        plsc.subcore_barrier()
        col0 = (c * 4 + rr) * 32

        def build(p):
            mbuf, abuf, pay = mbufs[p], abufs[p], pays[p]

            def row(r, _):
                rs = _splat(r)
                d0 = plsc.load_gather(abuf, [rs, _splat(2)])
                d1 = plsc.load_gather(abuf, [rs, _splat(3)])
                d2 = plsc.load_gather(abuf, [rs, _splat(4)])
                for cg in range(2):
                    m = mbuf[r, pl.ds(cg * 16, 16)]
                    pay[r, pl.ds(cg * 16, 16)] = m * d0
                    pay[r, pl.ds(32 + cg * 16, 16)] = m * d1
                    pay[r, pl.ds(64 + cg * 16, 16)] = m * d2
                return 0

            lax.fori_loop(0, CH, row, 0, unroll=4)

        _pipe_scatter_round(idx_hbm, mix_hbm, (col0, 32), att_hbm, acc,
                            base, (E // NS) // CH, ivs, mbufs, abufs, pays,
                            insems, scatsems, build)
        plsc.subcore_barrier()

        for oi in range(8):
            @pl.when(jnp.logical_and(c * 4 + rr == oi, s < 10))
            def _(oref=outs[oi]):
                pltpu.sync_copy(acc.at[pl.ds(s * 1000, 1000)],
                                oref.at[pl.ds(s * 1000, 1000)])
        plsc.subcore_barrier()


def _node_body(h_ref, x_ref, v_ref, sa_ref, sb_ref,
               hsa_ref, hsb_ref, hsc_ref, hsd_ref,
               s0_ref, s1_ref, s2_ref, s3_ref, s4_ref, s5_ref, s6_ref, s7_ref,
               va_ref, vb_ref,
               wp1_ref, bp1_ref, wp2_ref, bp2_ref,
               wn1h_ref, wn1s0_ref, wn1s1_ref, wn1sp_ref, bn1_ref,
               wn2_ref, bn2_ref, wv1_ref, bv1_ref, wv2_ref,
               hu_ref, xu_ref, vu_ref):
    h = h_ref[...]
    ssum = sa_ref[...] + sb_ref[...]
    cnt = ssum[:, 5:6]
    icnt = 1.0 / jnp.maximum(cnt, 1.0)
    schunks = [s0_ref[...], s1_ref[...], s2_ref[...], s3_ref[...],
               s4_ref[...], s5_ref[...], s6_ref[...], s7_ref[...]]
    norm_sq = jnp.zeros_like(h, shape=(h.shape[0], COEFF))
    for k in range(3):
        sk = jnp.concatenate([sc[:, 32 * k:32 * (k + 1)] for sc in schunks], axis=1)
        skm = sk * icnt
        norm_sq = norm_sq + skm * skm
    hsp = _silu(jnp.dot(norm_sq, wp1_ref[...], preferred_element_type=jnp.float32) + bp1_ref[...])
    hsp = _silu(jnp.dot(hsp, wp2_ref[...], preferred_element_type=jnp.float32) + bp2_ref[...])
    hs0 = jnp.concatenate([hsa_ref[...], hsb_ref[...]], axis=1)
    hs1 = jnp.concatenate([hsc_ref[...], hsd_ref[...]], axis=1)
    pre = (jnp.dot(h, wn1h_ref[...], preferred_element_type=jnp.float32)
           + jnp.dot(hs0, wn1s0_ref[...], preferred_element_type=jnp.float32)
           + jnp.dot(hs1, wn1s1_ref[...], preferred_element_type=jnp.float32)
           + jnp.dot(hsp, wn1sp_ref[...], preferred_element_type=jnp.float32)
           + bn1_ref[...])
    hu = h + _silu(jnp.dot(_silu(pre), wn2_ref[...], preferred_element_type=jnp.float32) + bn2_ref[...])
    hu_ref[...] = hu
    g = _silu(jnp.dot(h, wv1_ref[...], preferred_element_type=jnp.float32) + bv1_ref[...])
    gate = 2.0 / (1.0 + jnp.exp(-jnp.dot(g, wv2_ref[...], preferred_element_type=jnp.float32)[:, :1]))
    dv = (va_ref[...][:, :3] + vb_ref[...][:, :3]) * icnt
    vu = gate * v_ref[...] + dv
    vu_ref[...] = vu
    xu_ref[...] = x_ref[...] + vu


def _node_finalize(h, x, v, sa, sb, hs, schunks, va, vb, wp1, bp1, wp2,
                   bp2, wn1h, wn1s0, wn1s1, wn1sp, bn1, wn2, bn2, wv1, bv1, wv2):
    nb = 2000
    full = lambda r, c: pl.BlockSpec((r, c), lambda i: (0, 0))
    blk = lambda c: pl.BlockSpec((nb, c), lambda i: (i, 0))
    return pl.pallas_call(
        _node_body,
        grid=(N // nb,),
        in_specs=[
            blk(D), blk(3), blk(3), blk(8), blk(8),
            blk(64), blk(64), blk(64), blk(64),
            blk(96), blk(96), blk(96), blk(96),
            blk(96), blk(96), blk(96), blk(96),
            blk(8), blk(8),
            full(COEFF, D), full(1, D), full(D, D), full(1, D),
            full(D, D), full(D, D), full(D, D), full(D, D), full(1, D),
            full(D, D), full(1, D), full(D, D), full(1, D), full(D, 8),
        ],
        out_specs=[blk(D), blk(3), blk(3)],
        out_shape=[
            jax.ShapeDtypeStruct((N, D), jnp.float32),
            jax.ShapeDtypeStruct((N, 3), jnp.float32),
            jax.ShapeDtypeStruct((N, 3), jnp.float32),
        ],
    )(h, x, v, sa, sb, *hs, *schunks, va, vb, wp1, bp1, wp2, bp2,
      wn1h, wn1s0, wn1s1, wn1sp, bn1, wn2, bn2, wv1, bv1, wv2)


def kernel(h, x, v, pairlist, W_in, b_in, W_e1, b_e1, W_e2, b_e2, W_att, b_att,
           W_mix, W_v, W_n1, b_n1, W_n2, b_n2, W_p1, b_p1, W_p2, b_p2,
           W_vel1, b_vel1, W_vel2):
    f32 = jnp.float32
    idx_i = pairlist[0].astype(jnp.int32)
    idx_j = pairlist[1].astype(jnp.int32)

    z14 = jnp.zeros((D, PW - D - NRBF), f32)
    wi = jnp.concatenate([W_e1[:D], W_in[:D], z14], axis=1)
    wj = jnp.concatenate([W_e1[D:2 * D], W_in[D:], z14], axis=1)
    xpad = jnp.zeros((N, PW), f32).at[:, XC:XC + 3].set(x)
    xpi, xpj = -xpad, xpad
    we1f = jnp.zeros((PW - D, D), f32).at[:NRBF].set(W_e1[2 * D:2 * D + NRBF])
    wd = W_e1[2 * D + NRBF:2 * D + NRBF + 1]
    xpi = xpi.at[:, D:D + NRBF].add(0.5 * b_in[None, :])
    xpj = xpj.at[:, D:D + NRBF].add(0.5 * b_in[None, :])
    be1 = b_e1[None, :]
    be2 = b_e2[None, :]
    watt8 = jnp.zeros((D, 8), f32).at[:, :NH].set(W_att)
    batt8 = jnp.zeros((1, 8), f32).at[0, :NH].set(b_att)
    wm0 = W_mix[0::2]
    wm1 = W_mix[1::2]
    wv8 = jnp.zeros((COEFF, 8), f32).at[:, :1].set(W_v)
    wn1h = W_n1[:D]
    wn1sem = W_n1[D:D + COEFF].reshape(D, NH, D).transpose(1, 0, 2)
    wn1s0, wn1s1 = wn1sem[0], wn1sem[1]
    wn1sp = W_n1[D + COEFF:]
    wvel2_8 = jnp.zeros((D, 8), f32).at[:, :1].set(W_vel2)
    zeros_big = jnp.zeros((N, 192), f32)

    pi, pj = _nodeproj(h, xpi, xpj, wi, wj)
    elin = _edge_gather(pi, pj, idx_i, idx_j)
    he, ex = _edge_a(elin, we1f, wd, be1, W_e2, be2, watt8, batt8)
    sa, sb = _scatter_rows8(ex, idx_i, zeros_big)
    att = _edge_att(ex, sa, sb, idx_i)
    mix, vrow = _edge_b(he, att, wm0, wm1, wv8)
    hs = _scatter_hsem(he, att, idx_i, zeros_big)
    schunks = _scatter_smat(mix, att, idx_i, zeros_big)
    va, vb = _scatter_rows8(vrow, idx_i, zeros_big)
    hu, xu, vu = _node_finalize(
        h, x, v, sa, sb, list(hs), list(schunks), va, vb,
        W_p1, b_p1[None, :], W_p2, b_p2[None, :],
        wn1h, wn1s0, wn1s1, wn1sp, b_n1[None, :], W_n2, b_n2[None, :],
        W_vel1, b_vel1[None, :], wvel2_8)
    return (hu, xu, vu)
